# Initial kernel scaffold; baseline (speedup 1.0000x reference)
import jax
import jax.numpy as jnp
from jax.experimental import pallas as pl


def kernel(inputs):
    raise NotImplementedError("write your pallas kernel here")


# speedup vs baseline: 4.2552x; 4.2552x over previous
import functools

import jax
import jax.numpy as jnp
from jax import lax
from jax.experimental import pallas as pl
from jax.experimental.pallas import tpu as pltpu
from jax.experimental.pallas import tpu_sc as plsc

_B, _T, _K, _C = 32, 2000, 7, 128
_TC = 500
_NG = _T // _TC
_ROWS = _TC + _K - 1


def _conv_input_body(in_hbm, out_hbm, buf0, buf1, gsem, ssem0, ssem1):
    b = lax.axis_index("s") * 2 + lax.axis_index("c")
    bufs = (buf0, buf1)
    ssems = (ssem0, ssem1)

    zvec = jnp.zeros((16,), jnp.float32)
    for r in range(_K - 1):
        for c0 in range(0, _C, 16):
            buf0[r, pl.ds(c0, 16)] = zvec

    def start_gather(g):
        buf = bufs[g % 2]
        if g == 0:
            return pltpu.async_copy(
                in_hbm.at[b, pl.ds(0, _TC)], buf.at[pl.ds(_K - 1, _TC)], gsem)
        return pltpu.async_copy(
            in_hbm.at[b, pl.ds(g * _TC - (_K - 1), _ROWS)], buf, gsem)

    gather = start_gather(0)
    pending = [None, None]
    for g in range(_NG):
        buf = bufs[g % 2]
        ssem = ssems[g % 2]
        gather.wait()
        if g + 1 < _NG:
            nxt = (g + 1) % 2
            if pending[nxt] is not None:
                for d in pending[nxt]:
                    d.wait()
                pending[nxt] = None
            gather = start_gather(g + 1)
        descs = []
        for j in range(_K):
            descs.append(pltpu.async_copy(
                buf.at[pl.ds(j, _TC)],
                out_hbm.at[b, pl.ds(g * _TC, _TC), j], ssem))
        pending[g % 2] = descs
    for p in pending:
        if p is not None:
            for d in p:
                d.wait()


@functools.partial(
    pl.kernel,
    out_type=jax.ShapeDtypeStruct((_B, _T, _K, _C), jnp.float32),
    mesh=plsc.VectorSubcoreMesh(core_axis_name="c", subcore_axis_name="s"),
    compiler_params=pltpu.CompilerParams(use_tc_tiling_on_sc=False),
    scratch_types=[
        pltpu.VMEM((_ROWS, _C), jnp.float32),
        pltpu.VMEM((_ROWS, _C), jnp.float32),
        pltpu.SemaphoreType.DMA,
        pltpu.SemaphoreType.DMA,
        pltpu.SemaphoreType.DMA,
    ],
)
def _conv_input(in_hbm, out_hbm, buf0, buf1, gsem, ssem0, ssem1):
    _conv_input_body(in_hbm, out_hbm, buf0, buf1, gsem, ssem0, ssem1)


def kernel(inputs):
    return _conv_input(inputs)


# speedup vs baseline: 9.1357x; 2.1470x over previous
import functools

import jax
import jax.numpy as jnp
from jax import lax
from jax.experimental import pallas as pl
from jax.experimental.pallas import tpu as pltpu
from jax.experimental.pallas import tpu_sc as plsc

_B, _T, _K, _C = 32, 2000, 7, 128
_TCH = 40
_NCH = _T // _TCH
_HALO = 8
_RIN = _TCH + _HALO


def _expand(src, dst):
    @plsc.parallel_loop(0, _TCH, unroll=4)
    def row(u):
        for j in range(_K):
            for c0 in range(0, _C, 16):
                dst[u, j, pl.ds(c0, 16)] = src[u + j + (_HALO - _K + 1),
                                               pl.ds(c0, 16)]


def _conv_input_body(in_hbm, out_hbm, in0, in1, o0, o1, gs0, gs1, ss0, ss1):
    b = lax.axis_index("s") * 2 + lax.axis_index("c")

    def gather_src(g):
        return in_hbm.at[b, pl.ds(pl.multiple_of(g * _TCH - _HALO, 8), _RIN)]

    def scatter_dst(g):
        return out_hbm.at[b, pl.ds(g * _TCH, _TCH)]

    def start_gather(g, buf, sem):
        return pltpu.async_copy(gather_src(g), buf, sem)

    def wait_gather(buf, sem):
        pltpu.make_async_copy(gather_src(1), buf, sem).wait()

    def drain_scatter(buf, sem):
        pltpu.make_async_copy(buf, scatter_dst(0), sem).wait()

    zvec = jnp.zeros((16,), jnp.float32)
    for r in range(_HALO):
        for c0 in range(0, _C, 16):
            in0[r, pl.ds(c0, 16)] = zvec
    g0 = pltpu.async_copy(in_hbm.at[b, pl.ds(0, _TCH)],
                          in0.at[pl.ds(_HALO, _TCH)], gs0)
    g0.wait()
    start_gather(1, in1, gs1)
    _expand(in0, o0)
    pltpu.async_copy(o0, scatter_dst(0), ss0)

    wait_gather(in1, gs1)
    start_gather(2, in0, gs0)
    _expand(in1, o1)
    pltpu.async_copy(o1, scatter_dst(1), ss1)

    def pair(p, carry):
        g_even = p * 2
        drain_scatter(o0, ss0)
        wait_gather(in0, gs0)
        start_gather(g_even + 1, in1, gs1)
        _expand(in0, o0)
        pltpu.async_copy(o0, scatter_dst(g_even), ss0)

        g_odd = g_even + 1
        drain_scatter(o1, ss1)
        wait_gather(in1, gs1)
        start_gather(g_odd + 1, in0, gs0)
        _expand(in1, o1)
        pltpu.async_copy(o1, scatter_dst(g_odd), ss1)
        return carry

    lax.fori_loop(1, _NCH // 2 - 1, pair, 0)

    drain_scatter(o0, ss0)
    wait_gather(in0, gs0)
    g49 = start_gather(_NCH - 1, in1, gs1)
    _expand(in0, o0)
    pltpu.async_copy(o0, scatter_dst(_NCH - 2), ss0)

    drain_scatter(o1, ss1)
    g49.wait()
    _expand(in1, o1)
    pltpu.async_copy(o1, scatter_dst(_NCH - 1), ss1)

    drain_scatter(o0, ss0)
    drain_scatter(o1, ss1)


@functools.partial(
    pl.kernel,
    out_type=jax.ShapeDtypeStruct((_B, _T, _K, _C), jnp.float32),
    mesh=plsc.VectorSubcoreMesh(core_axis_name="c", subcore_axis_name="s"),
    scratch_types=[
        pltpu.VMEM((_RIN, _C), jnp.float32),
        pltpu.VMEM((_RIN, _C), jnp.float32),
        pltpu.VMEM((_TCH, _K, _C), jnp.float32),
        pltpu.VMEM((_TCH, _K, _C), jnp.float32),
        pltpu.SemaphoreType.DMA,
        pltpu.SemaphoreType.DMA,
        pltpu.SemaphoreType.DMA,
        pltpu.SemaphoreType.DMA,
    ],
)
def _conv_input(in_hbm, out_hbm, in0, in1, o0, o1, gs0, gs1, ss0, ss1):
    _conv_input_body(in_hbm, out_hbm, in0, in1, o0, o1, gs0, gs1, ss0, ss1)


def kernel(inputs):
    return _conv_input(inputs)


# speedup vs baseline: 24.9021x; 2.7258x over previous
import functools

import jax
import jax.numpy as jnp
from jax import lax
from jax.experimental import pallas as pl
from jax.experimental.pallas import tpu as pltpu
from jax.experimental.pallas import tpu_sc as plsc

_B, _T, _K, _C = 32, 2000, 7, 128
_TC = 250
_NG = _T // _TC
_ROWS = _TC + _K - 1


def _conv_input_body(in_hbm, out_hbm, buf0, buf1, gsem, ssem0, ssem1):
    b = lax.axis_index("s") * 2 + lax.axis_index("c")
    bufs = (buf0, buf1)
    ssems = (ssem0, ssem1)

    zvec = jnp.zeros((16,), jnp.float32)
    for r in range(_K - 1):
        for c0 in range(0, _C, 16):
            buf0[r, pl.ds(c0, 16)] = zvec

    def start_gather(g):
        buf = bufs[g % 2]
        if g == 0:
            return pltpu.async_copy(
                in_hbm.at[b, pl.ds(0, _TC)], buf.at[pl.ds(_K - 1, _TC)], gsem)
        return pltpu.async_copy(
            in_hbm.at[b, pl.ds(g * _TC - (_K - 1), _ROWS)], buf, gsem)

    gather = start_gather(0)
    pending = [None, None]
    for g in range(_NG):
        buf = bufs[g % 2]
        ssem = ssems[g % 2]
        gather.wait()
        if g + 1 < _NG:
            nxt = (g + 1) % 2
            if pending[nxt] is not None:
                for d in pending[nxt]:
                    d.wait()
                pending[nxt] = None
            gather = start_gather(g + 1)
        descs = []
        for j in range(_K):
            descs.append(pltpu.async_copy(
                buf.at[pl.ds(j, _TC)],
                out_hbm.at[b, j, pl.ds(g * _TC, _TC)], ssem))
        pending[g % 2] = descs
    for p in pending:
        if p is not None:
            for d in p:
                d.wait()


@functools.partial(
    pl.kernel,
    out_type=jax.ShapeDtypeStruct((_B, _K, _T, _C), jnp.float32),
    mesh=plsc.VectorSubcoreMesh(core_axis_name="c", subcore_axis_name="s"),
    compiler_params=pltpu.CompilerParams(use_tc_tiling_on_sc=False),
    scratch_types=[
        pltpu.VMEM((_ROWS, _C), jnp.float32),
        pltpu.VMEM((_ROWS, _C), jnp.float32),
        pltpu.SemaphoreType.DMA,
        pltpu.SemaphoreType.DMA,
        pltpu.SemaphoreType.DMA,
    ],
)
def _conv_input(in_hbm, out_hbm, buf0, buf1, gsem, ssem0, ssem1):
    _conv_input_body(in_hbm, out_hbm, buf0, buf1, gsem, ssem0, ssem1)


def kernel(inputs):
    return jnp.transpose(_conv_input(inputs), (0, 2, 1, 3))


# speedup vs baseline: 26.0928x; 1.0478x over previous
import functools

import jax
import jax.numpy as jnp
from jax import lax
from jax.experimental import pallas as pl
from jax.experimental.pallas import tpu as pltpu
from jax.experimental.pallas import tpu_sc as plsc

_B, _T, _K, _C = 32, 2000, 7, 128
_TC = 500
_NG = _T // _TC
_ROWS = _TC + _K - 1


def _conv_input_body(in_hbm, out_hbm, buf0, buf1, gsem, ssem0, ssem1):
    b = lax.axis_index("s") * 2 + lax.axis_index("c")
    bufs = (buf0, buf1)
    ssems = (ssem0, ssem1)

    zvec = jnp.zeros((16,), jnp.float32)
    for r in range(_K - 1):
        for c0 in range(0, _C, 16):
            buf0[r, pl.ds(c0, 16)] = zvec

    def start_gather(g):
        buf = bufs[g % 2]
        if g == 0:
            return pltpu.async_copy(
                in_hbm.at[b, pl.ds(0, _TC)], buf.at[pl.ds(_K - 1, _TC)], gsem)
        return pltpu.async_copy(
            in_hbm.at[b, pl.ds(g * _TC - (_K - 1), _ROWS)], buf, gsem)

    gather = start_gather(0)
    pending = [None, None]
    for g in range(_NG):
        buf = bufs[g % 2]
        ssem = ssems[g % 2]
        gather.wait()
        if g + 1 < _NG:
            nxt = (g + 1) % 2
            if pending[nxt] is not None:
                for d in pending[nxt]:
                    d.wait()
                pending[nxt] = None
            gather = start_gather(g + 1)
        descs = []
        for j in range(_K):
            descs.append(pltpu.async_copy(
                buf.at[pl.ds(j, _TC)],
                out_hbm.at[b, j, pl.ds(g * _TC, _TC)], ssem))
        pending[g % 2] = descs
    for p in pending:
        if p is not None:
            for d in p:
                d.wait()


@functools.partial(
    pl.kernel,
    out_type=jax.ShapeDtypeStruct((_B, _K, _T, _C), jnp.float32),
    mesh=plsc.VectorSubcoreMesh(core_axis_name="c", subcore_axis_name="s"),
    compiler_params=pltpu.CompilerParams(use_tc_tiling_on_sc=False),
    scratch_types=[
        pltpu.VMEM((_ROWS, _C), jnp.float32),
        pltpu.VMEM((_ROWS, _C), jnp.float32),
        pltpu.SemaphoreType.DMA,
        pltpu.SemaphoreType.DMA,
        pltpu.SemaphoreType.DMA,
    ],
)
def _conv_input(in_hbm, out_hbm, buf0, buf1, gsem, ssem0, ssem1):
    _conv_input_body(in_hbm, out_hbm, buf0, buf1, gsem, ssem0, ssem1)


def kernel(inputs):
    return jnp.transpose(_conv_input(inputs), (0, 2, 1, 3))
